# baseline (speedup 1.0000x reference)
import functools

import jax
import jax.numpy as jnp
from jax import lax
from jax.experimental import pallas as pl
from jax.experimental.pallas import tpu as pltpu
from jax.experimental.pallas import tpu_sc as plsc

NUM_CORES = 2
NUM_SUBCORES = 16
NW = NUM_CORES * NUM_SUBCORES
LANES = 16


@functools.lru_cache(maxsize=None)
def _build_gather(B: int, S: int, V: int, D: int):
    assert B % (NW * 128) == 0 and D % 8 == 0
    bt_per_w = B // (NW * 128)
    bpw = bt_per_w * 128
    nunits = S * bt_per_w
    ET = D // 8

    mesh = plsc.VectorSubcoreMesh(core_axis_name="c", subcore_axis_name="s")

    @functools.partial(
        pl.kernel,
        out_type=jax.ShapeDtypeStruct((S, ET, B // 128, 8, 128), jnp.float32),
        mesh=mesh,
        scratch_types=[
            pltpu.VMEM((S, bpw), jnp.int32),
            pltpu.VMEM((2, 128, D), jnp.float32),
            pltpu.VMEM((2, ET, 8, 128), jnp.float32),
        ] + [pltpu.SemaphoreType.DMA] * 4,
        compiler_params=pltpu.CompilerParams(
            use_tc_tiling_on_sc=False, needs_layout_passes=False),
    )
    def gather_kernel(xt_hbm, table_hbm, out_hbm, idx_v, rows_v, tile_v,
                      gsem0, gsem1, wsem0, wsem1):
        wid = lax.axis_index("s") * NUM_CORES + lax.axis_index("c")
        b0 = wid * bpw
        gsems = (gsem0, gsem1)
        wsems = (wsem0, wsem1)
        pltpu.sync_copy(xt_hbm.at[:, pl.ds(b0, bpw)], idx_v)

        def issue_gather(i, p):
            s, u = i // bt_per_w, lax.rem(i, bt_per_w)
            pltpu.async_copy(
                table_hbm.at[idx_v.at[s, pl.ds(u * 128, 128)]],
                rows_v.at[p], gsems[p],
            )

        def wait_gather(p):
            pltpu.make_async_copy(
                table_hbm.at[pl.ds(0, 128)], rows_v.at[p], gsems[p]
            ).wait()

        def transpose(p):
            for c in range(D):
                for k in range(128 // LANES):
                    row_ids = lax.iota(jnp.int32, LANES) + (k * LANES)
                    col_ids = jnp.full((LANES,), c, jnp.int32)
                    vals = plsc.load_gather(rows_v.at[p], [row_ids, col_ids])
                    tile_v[p, c // 8, c % 8, pl.ds(k * LANES, LANES)] = vals

        def issue_writes(i, p):
            s, u = i // bt_per_w, lax.rem(i, bt_per_w)
            bc = wid * bt_per_w + u
            for er in range(ET):
                pltpu.async_copy(
                    tile_v.at[p, er], out_hbm.at[s, er, bc], wsems[p],
                )

        def wait_writes(p):
            for er in range(ET):
                pltpu.make_async_copy(
                    tile_v.at[p, er], out_hbm.at[0, er, 0], wsems[p]
                ).wait()

        issue_gather(0, 0)
        issue_gather(1, 1)

        @pl.loop(0, nunits)
        def _(i):
            par = lax.rem(i, 2)

            def _unit(p):
                wait_gather(p)

                @pl.when(i >= 2)
                def _():
                    wait_writes(p)

                transpose(p)

                @pl.when(i + 2 < nunits)
                def _():
                    issue_gather(i + 2, p)

                issue_writes(i, p)

            @pl.when(par == 0)
            def _():
                _unit(0)

            @pl.when(par == 1)
            def _():
                _unit(1)

        wait_writes(0)
        wait_writes(1)

    return gather_kernel


def kernel(x, weight):
    V, D = weight.shape
    B, S = x.shape
    xt = jnp.transpose(x)
    out5 = _build_gather(B, S, V, D)(xt, weight)
    out = jnp.transpose(out5, (2, 4, 0, 1, 3)).reshape(B, S, D)
    return out


# speedup vs baseline: 1.7201x; 1.7201x over previous
import functools

import jax
import jax.numpy as jnp
from jax import lax
from jax.experimental import pallas as pl
from jax.experimental.pallas import tpu as pltpu
from jax.experimental.pallas import tpu_sc as plsc

NUM_CORES = 2
NUM_SUBCORES = 16
NW = NUM_CORES * NUM_SUBCORES
LANES = 16


@functools.lru_cache(maxsize=None)
def _build_gather(B: int, S: int, V: int, D: int):
    assert B % (NW * 128) == 0 and D % 8 == 0
    bt_per_w = B // (NW * 128)
    bpw = bt_per_w * 128
    nunits = S * bt_per_w
    ET = D // 8

    mesh = plsc.VectorSubcoreMesh(core_axis_name="c", subcore_axis_name="s")

    @functools.partial(
        pl.kernel,
        out_type=jax.ShapeDtypeStruct((S, ET, B // 128, 8, 128), jnp.float32),
        mesh=mesh,
        scratch_types=[
            pltpu.VMEM((S, bpw), jnp.int32),
            pltpu.VMEM((2, 128, D), jnp.float32),
            pltpu.VMEM((2, ET, 8, 129), jnp.float32),
        ] + [pltpu.SemaphoreType.DMA] * 4,
        compiler_params=pltpu.CompilerParams(
            use_tc_tiling_on_sc=False, needs_layout_passes=False),
    )
    def gather_kernel(xt_hbm, table_hbm, out_hbm, idx_v, rows_v, tile_v,
                      gsem0, gsem1, wsem0, wsem1):
        wid = lax.axis_index("s") * NUM_CORES + lax.axis_index("c")
        b0 = wid * bpw
        gsems = (gsem0, gsem1)
        wsems = (wsem0, wsem1)
        pltpu.sync_copy(xt_hbm.at[:, pl.ds(b0, bpw)], idx_v)

        def issue_gather(i, p):
            s, u = i // bt_per_w, lax.rem(i, bt_per_w)
            pltpu.async_copy(
                table_hbm.at[idx_v.at[s, pl.ds(u * 128, 128)]],
                rows_v.at[p], gsems[p],
            )

        def wait_gather(p):
            pltpu.make_async_copy(
                table_hbm.at[pl.ds(0, 128)], rows_v.at[p], gsems[p]
            ).wait()

        def transpose(p):
            lane = lax.iota(jnp.int32, LANES)
            er_base = lax.shift_right_logical(lane, 3)
            es_ids = lax.bitwise_and(lane, 7)
            for q in range(D // LANES):
                er_ids = er_base + (2 * q)
                for bl in range(128):
                    vals = rows_v[p, bl, pl.ds(q * LANES, LANES)]
                    bl_ids = jnp.full((LANES,), bl, jnp.int32)
                    plsc.store_scatter(
                        tile_v.at[p], [er_ids, es_ids, bl_ids], vals)

        def issue_writes(i, p):
            s, u = i // bt_per_w, lax.rem(i, bt_per_w)
            bc = wid * bt_per_w + u
            for er in range(ET):
                pltpu.async_copy(
                    tile_v.at[p, er, :, pl.ds(0, 128)],
                    out_hbm.at[s, er, bc], wsems[p],
                )

        def wait_writes(p):
            for er in range(ET):
                pltpu.make_async_copy(
                    tile_v.at[p, er, :, pl.ds(0, 128)],
                    out_hbm.at[0, er, 0], wsems[p]
                ).wait()

        issue_gather(0, 0)
        issue_gather(1, 1)

        @pl.loop(0, nunits)
        def _(i):
            par = lax.rem(i, 2)

            def _unit(p):
                wait_gather(p)

                @pl.when(i >= 2)
                def _():
                    wait_writes(p)

                transpose(p)

                @pl.when(i + 2 < nunits)
                def _():
                    issue_gather(i + 2, p)

                issue_writes(i, p)

            @pl.when(par == 0)
            def _():
                _unit(0)

            @pl.when(par == 1)
            def _():
                _unit(1)

        wait_writes(0)
        wait_writes(1)

    return gather_kernel


def kernel(x, weight):
    V, D = weight.shape
    B, S = x.shape
    xt = jnp.transpose(x)
    out5 = _build_gather(B, S, V, D)(xt, weight)
    out = jnp.transpose(out5, (2, 4, 0, 1, 3)).reshape(B, S, D)
    return out


# speedup vs baseline: 2.0065x; 1.1665x over previous
import functools

import jax
import jax.numpy as jnp
from jax import lax
from jax.experimental import pallas as pl
from jax.experimental.pallas import tpu as pltpu
from jax.experimental.pallas import tpu_sc as plsc

NUM_CORES = 2
NUM_SUBCORES = 16
NW = NUM_CORES * NUM_SUBCORES
LANES = 16


@functools.lru_cache(maxsize=None)
def _build_gather(B: int, S: int, V: int, D: int):
    assert B % (NW * 128) == 0 and D % 8 == 0
    bt_per_w = B // (NW * 128)
    bpw = bt_per_w * 128
    nunits = S * bt_per_w
    ET = D // 8

    mesh = plsc.VectorSubcoreMesh(core_axis_name="c", subcore_axis_name="s")

    @functools.partial(
        pl.kernel,
        out_type=jax.ShapeDtypeStruct((S, ET, B // 128, 8, 128), jnp.float32),
        mesh=mesh,
        scratch_types=[
            pltpu.VMEM((S, bpw), jnp.int32),
            pltpu.VMEM((2, 128, D), jnp.float32),
            pltpu.VMEM((2, ET, 8, 129), jnp.float32),
        ] + [pltpu.SemaphoreType.DMA] * 4,
        compiler_params=pltpu.CompilerParams(
            use_tc_tiling_on_sc=False, needs_layout_passes=False),
    )
    def gather_kernel(xt_hbm, table_hbm, out_hbm, idx_v, rows_v, tile_v,
                      gsem0, gsem1, wsem0, wsem1):
        wid = lax.axis_index("s") * NUM_CORES + lax.axis_index("c")
        b0 = wid * bpw
        gsems = (gsem0, gsem1)
        wsems = (wsem0, wsem1)
        pltpu.sync_copy(xt_hbm.at[:, pl.ds(b0, bpw)], idx_v)

        def issue_gather(i, p):
            s, u = i // bt_per_w, lax.rem(i, bt_per_w)
            pltpu.async_copy(
                table_hbm.at[idx_v.at[s, pl.ds(u * 128, 128)]],
                rows_v.at[p], gsems[p],
            )

        def wait_gather(p):
            pltpu.make_async_copy(
                table_hbm.at[pl.ds(0, 128)], rows_v.at[p], gsems[p]
            ).wait()

        def transpose(p):
            lane = lax.iota(jnp.int32, LANES)
            er_base = lax.shift_right_logical(lane, 3)
            es_ids = lax.bitwise_and(lane, 7)
            for q in range(D // LANES):
                er_ids = er_base + (2 * q)
                for bl0 in range(0, 128, 8):
                    vals = [rows_v[p, bl0 + j, pl.ds(q * LANES, LANES)]
                            for j in range(8)]
                    for j in range(8):
                        bl_ids = jnp.full((LANES,), bl0 + j, jnp.int32)
                        plsc.store_scatter(
                            tile_v.at[p], [er_ids, es_ids, bl_ids], vals[j])

        def issue_writes(i, p):
            s, u = i // bt_per_w, lax.rem(i, bt_per_w)
            bc = wid * bt_per_w + u
            for er in range(ET):
                pltpu.async_copy(
                    tile_v.at[p, er, :, pl.ds(0, 128)],
                    out_hbm.at[s, er, bc], wsems[p],
                )

        def wait_writes(p):
            for er in range(ET):
                pltpu.make_async_copy(
                    tile_v.at[p, er, :, pl.ds(0, 128)],
                    out_hbm.at[0, er, 0], wsems[p]
                ).wait()

        issue_gather(0, 0)
        issue_gather(1, 1)

        @pl.loop(0, nunits)
        def _(i):
            par = lax.rem(i, 2)

            def _unit(p):
                wait_gather(p)

                @pl.when(i >= 2)
                def _():
                    wait_writes(p)

                transpose(p)

                @pl.when(i + 2 < nunits)
                def _():
                    issue_gather(i + 2, p)

                issue_writes(i, p)

            @pl.when(par == 0)
            def _():
                _unit(0)

            @pl.when(par == 1)
            def _():
                _unit(1)

        wait_writes(0)
        wait_writes(1)

    return gather_kernel


def kernel(x, weight):
    V, D = weight.shape
    B, S = x.shape
    xt = jnp.transpose(x)
    out5 = _build_gather(B, S, V, D)(xt, weight)
    out = jnp.transpose(out5, (2, 4, 0, 1, 3)).reshape(B, S, D)
    return out


# speedup vs baseline: 2.0070x; 1.0003x over previous
import functools

import jax
import jax.numpy as jnp
from jax import lax
from jax.experimental import pallas as pl
from jax.experimental.pallas import tpu as pltpu
from jax.experimental.pallas import tpu_sc as plsc

NUM_CORES = 2
NUM_SUBCORES = 16
NW = NUM_CORES * NUM_SUBCORES
LANES = 16


@functools.lru_cache(maxsize=None)
def _build_gather(B: int, S: int, V: int, D: int):
    assert B % (NW * 128) == 0 and D % 8 == 0
    bt_per_w = B // (NW * 128)
    bpw = bt_per_w * 128
    nunits = S * bt_per_w
    ET = D // 8

    mesh = plsc.VectorSubcoreMesh(core_axis_name="c", subcore_axis_name="s")

    @functools.partial(
        pl.kernel,
        out_type=jax.ShapeDtypeStruct((S, ET, B // 128, 8, 128), jnp.float32),
        mesh=mesh,
        scratch_types=[
            pltpu.VMEM((S, bpw), jnp.int32),
            pltpu.VMEM((2, 128, D), jnp.float32),
            pltpu.VMEM((2, ET, 8, 129), jnp.float32),
        ] + [pltpu.SemaphoreType.DMA] * 4,
        compiler_params=pltpu.CompilerParams(
            use_tc_tiling_on_sc=False, needs_layout_passes=False),
    )
    def gather_kernel(xt_hbm, table_hbm, out_hbm, idx_v, rows_v, tile_v,
                      gsem0, gsem1, wsem0, wsem1):
        wid = lax.axis_index("s") * NUM_CORES + lax.axis_index("c")
        b0 = wid * bpw
        gsems = (gsem0, gsem1)
        wsems = (wsem0, wsem1)
        pltpu.sync_copy(xt_hbm.at[:, pl.ds(b0, bpw)], idx_v)

        def issue_gather(i, p):
            s, u = i // bt_per_w, lax.rem(i, bt_per_w)
            pltpu.async_copy(
                table_hbm.at[idx_v.at[s, pl.ds(u * 128, 128)]],
                rows_v.at[p], gsems[p],
            )

        def wait_gather(p):
            pltpu.make_async_copy(
                table_hbm.at[pl.ds(0, 128)], rows_v.at[p], gsems[p]
            ).wait()

        def transpose(p):
            lane = lax.iota(jnp.int32, LANES)
            er_base = lax.shift_right_logical(lane, 3)
            es_ids = lax.bitwise_and(lane, 7)
            for q in range(D // LANES):
                er_ids = er_base + (2 * q)
                for bl0 in range(0, 128, 8):
                    vals = [rows_v[p, bl0 + j, pl.ds(q * LANES, LANES)]
                            for j in range(8)]
                    for j in range(8):
                        bl_ids = jnp.full((LANES,), bl0 + j, jnp.int32)
                        plsc.store_scatter(
                            tile_v.at[p], [er_ids, es_ids, bl_ids], vals[j])

        def issue_writes(i, p):
            s, u = i // bt_per_w, lax.rem(i, bt_per_w)
            bc = wid * bt_per_w + u
            for er in range(ET):
                pltpu.async_copy(
                    tile_v.at[p, er, :, pl.ds(0, 128)],
                    out_hbm.at[s, er, bc], wsems[p],
                )

        def wait_writes(p):
            for er in range(ET):
                pltpu.make_async_copy(
                    tile_v.at[p, er, :, pl.ds(0, 128)],
                    out_hbm.at[0, er, 0], wsems[p]
                ).wait()

        issue_gather(0, 0)
        issue_gather(1, 1)

        @pl.loop(0, nunits)
        def _(i):
            par = lax.rem(i, 2)

            def _unit(p):
                wait_gather(p)

                @pl.when(i >= 2)
                def _():
                    wait_writes(p)

                transpose(p)

                @pl.when(i + 2 < nunits)
                def _():
                    issue_gather(i + 2, p)

                issue_writes(i, p)

            @pl.when(par == 0)
            def _():
                _unit(0)

            @pl.when(par == 1)
            def _():
                _unit(1)

        wait_writes(0)
        wait_writes(1)

    return gather_kernel


def kernel(x, weight):
    V, D = weight.shape
    B, S = x.shape
    xt = jnp.transpose(x)
    wlin = lax.optimization_barrier(weight.reshape(-1)).reshape(V, D)
    out5 = _build_gather(B, S, V, D)(xt, wlin)
    out = jnp.transpose(out5, (2, 4, 0, 1, 3)).reshape(B, S, D)
    return out
